# baseline (speedup 1.0000x reference)
import jax
import jax.numpy as jnp
from jax import lax
from jax.experimental import pallas as pl
from jax.experimental.pallas import tpu as pltpu
from jax.experimental.pallas import tpu_sc as plsc

B, C, K = 4096, 20, 5
D, VDIM = 64, 16

NC, NS = 2, 16
NW = NC * NS
CHUNK = 128

U_CH_W = (B * C) // (NW * CHUNK)
P_CH_W = B // (NW * CHUNK)
N_CH_W = (B * K) // (NW * CHUNK)


def _sc_gather_body(u_hbm, w_hbm, iu_hbm, ip_hbm, in_hbm,
                    gu_hbm, gp_hbm, gn_hbm,
                    iuv, ipv, inv, bufs, gs0, gs1, os0, os1):
    wid = lax.axis_index("s") * NC + lax.axis_index("c")
    pltpu.sync_copy(iu_hbm.at[pl.ds(wid * U_CH_W * CHUNK, U_CH_W * CHUNK)], iuv)
    pltpu.sync_copy(ip_hbm.at[pl.ds(wid * P_CH_W * CHUNK, P_CH_W * CHUNK)], ipv)
    pltpu.sync_copy(in_hbm.at[pl.ds(wid * N_CH_W * CHUNK, N_CH_W * CHUNK)], inv)

    chunks = []
    for j in range(U_CH_W):
        chunks.append((u_hbm, iuv.at[pl.ds(j * CHUNK, CHUNK)], gu_hbm,
                       (wid * U_CH_W + j) * CHUNK))
    for j in range(P_CH_W):
        chunks.append((w_hbm, ipv.at[pl.ds(j * CHUNK, CHUNK)], gp_hbm,
                       (wid * P_CH_W + j) * CHUNK))
    for j in range(N_CH_W):
        chunks.append((w_hbm, inv.at[pl.ds(j * CHUNK, CHUNK)], gn_hbm,
                       (wid * N_CH_W + j) * CHUNK))

    gsem = [gs0, gs1]
    osem = [os0, os1]
    gd = [None, None]
    od = [None, None]
    n = len(chunks)
    for j in range(n + 1):
        b = j % 2
        if j < n:
            tab, idx_row, _, _ = chunks[j]
            if od[b] is not None:
                od[b].wait()
            gd[b] = pltpu.async_copy(tab.at[idx_row], bufs.at[b], gsem[b])
        if j >= 1:
            pb = (j - 1) % 2
            _, _, out_ref, base = chunks[j - 1]
            gd[pb].wait()
            od[pb] = pltpu.async_copy(bufs.at[pb], out_ref.at[pl.ds(base, CHUNK)],
                                      osem[pb])
    od[(n - 1) % 2].wait()
    od[(n - 2) % 2].wait()


def _sc_gather(U, W, iu, ip, in_):
    return pl.kernel(
        _sc_gather_body,
        out_type=(
            jax.ShapeDtypeStruct((B * C, D), jnp.float32),
            jax.ShapeDtypeStruct((B, D), jnp.float32),
            jax.ShapeDtypeStruct((B * K, D), jnp.float32),
        ),
        mesh=plsc.VectorSubcoreMesh(core_axis_name="c", subcore_axis_name="s"),
        compiler_params=pltpu.CompilerParams(use_tc_tiling_on_sc=False),
        scratch_types=[
            pltpu.VMEM((U_CH_W * CHUNK,), jnp.int32),
            pltpu.VMEM((P_CH_W * CHUNK,), jnp.int32),
            pltpu.VMEM((N_CH_W * CHUNK,), jnp.int32),
            pltpu.VMEM((2, CHUNK, D), jnp.float32),
            pltpu.SemaphoreType.DMA,
            pltpu.SemaphoreType.DMA,
            pltpu.SemaphoreType.DMA,
            pltpu.SemaphoreType.DMA,
        ],
    )(U, W, iu, ip, in_)


BB = 256
GRID = B // BB


def _log_sigmoid(x):
    return jnp.minimum(x, 0.0) - jnp.log(1.0 + jnp.exp(-jnp.abs(x)))


def _tc_body(vp_ref, gu_ref, gp_ref, gn_ref, out_ref):
    @pl.when(pl.program_id(0) == 0)
    def _init():
        out_ref[0, 0] = 0.0

    vp = vp_ref[...]
    vp2 = vp * vp
    dn = (((1,), (1,)), ((), ()))
    S = jnp.zeros((BB, D), jnp.float32)
    acc = jnp.zeros((BB, 1), jnp.float32)
    for c in range(C):
        ec = gu_ref[:, c, :]
        t = lax.dot_general(ec, vp, dn, preferred_element_type=jnp.float32)
        t2 = lax.dot_general(ec * ec, vp2, dn, preferred_element_type=jnp.float32)
        acc = acc + jnp.sum(t * t - t2, axis=1, keepdims=True)
        S = S + ec
    fm = 0.5 * acc
    pu = S + C * fm
    s2 = jnp.sum(pu * gp_ref[...], axis=1, keepdims=True)
    nsum = jnp.zeros((BB, D), jnp.float32)
    for k in range(K):
        nsum = nsum + gn_ref[:, k, :]
    ns2 = jnp.sum(nsum * pu, axis=1, keepdims=True)
    part = jnp.sum(_log_sigmoid(s2)) + jnp.sum(_log_sigmoid(-ns2))
    out_ref[0, 0] += part


def _tc_score(Vp, gu3, gp, gn3):
    return pl.pallas_call(
        _tc_body,
        grid=(GRID,),
        in_specs=[
            pl.BlockSpec((VDIM, D), lambda i: (0, 0)),
            pl.BlockSpec((BB, C, D), lambda i: (i, 0, 0)),
            pl.BlockSpec((BB, D), lambda i: (i, 0)),
            pl.BlockSpec((BB, K, D), lambda i: (i, 0, 0)),
        ],
        out_specs=pl.BlockSpec((1, 1), lambda i: (0, 0),
                               memory_space=pltpu.SMEM),
        out_shape=jax.ShapeDtypeStruct((1, 1), jnp.float32),
    )(Vp, gu3, gp, gn3)


def kernel(pos_u, pos_w, neg_w, U, W, Vp):
    iu = pos_u.reshape(-1).astype(jnp.int32)
    ip = pos_w.reshape(-1).astype(jnp.int32)
    in_ = neg_w.reshape(-1).astype(jnp.int32)
    gu, gp, gn = _sc_gather(U, W, iu, ip, in_)
    out = _tc_score(Vp, gu.reshape(B, C, D), gp, gn.reshape(B, K, D))
    return -out[0, 0]
